# Initial kernel scaffold; baseline (speedup 1.0000x reference)
import jax
import jax.numpy as jnp
from jax.experimental import pallas as pl


def kernel(x, fltr, W00, V00, b00, W01, V01, b01, W10, V10, b10, W11, V11, b11):
    raise NotImplementedError("write your pallas kernel here")


# speedup vs baseline: 1.8615x; 1.8615x over previous
import jax
import jax.numpy as jnp
from jax.experimental import pallas as pl
from jax.experimental.pallas import tpu as pltpu

N = 10000
F = 128
C = 128
BM = 400
BX = 2000


def _prologue_kernel(x_ref, wcat_ref, vcat_ref, bcat_ref, p0_ref, s0_ref, s1_ref):
    x = x_ref[...]
    p0_ref[...] = jnp.dot(x, wcat_ref[...], preferred_element_type=jnp.float32)
    s = jnp.dot(x, vcat_ref[...], preferred_element_type=jnp.float32) + bcat_ref[...]
    s0_ref[...] = s[:, : 2 * C]
    s1_ref[...] = s[:, 2 * C :]


def _stage1_kernel(fltr_ref, p0_ref, s0_ref, w01_ref, w11_ref, p1_ref):
    z = jnp.dot(fltr_ref[...], p0_ref[...], preferred_element_type=jnp.float32)
    h = jnp.maximum(z + s0_ref[...], 0.0)
    a = jnp.dot(h[:, :C], w01_ref[...], preferred_element_type=jnp.float32)
    b = jnp.dot(h[:, C:], w11_ref[...], preferred_element_type=jnp.float32)
    p1_ref[...] = jnp.concatenate([a, b], axis=1)


def _stage2_kernel(fltr_ref, p1_ref, s1_ref, out_ref):
    z = jnp.dot(fltr_ref[...], p1_ref[...], preferred_element_type=jnp.float32)
    h = jnp.maximum(z + s1_ref[...], 0.0)
    out_ref[...] = 0.5 * (h[:, :C] + h[:, C:])


def kernel(x, fltr, W00, V00, b00, W01, V01, b01, W10, V10, b10, W11, V11, b11):
    wcat = jnp.concatenate([W00, W10], axis=1)
    vcat = jnp.concatenate([V00, V10, V01, V11], axis=1)
    bcat = jnp.concatenate([b00, b10, b01, b11]).reshape(1, 4 * C)

    p0, s0, s1 = pl.pallas_call(
        _prologue_kernel,
        grid=(N // BX,),
        in_specs=[
            pl.BlockSpec((BX, F), lambda i: (i, 0)),
            pl.BlockSpec((F, 2 * C), lambda i: (0, 0)),
            pl.BlockSpec((F, 4 * C), lambda i: (0, 0)),
            pl.BlockSpec((1, 4 * C), lambda i: (0, 0)),
        ],
        out_specs=[
            pl.BlockSpec((BX, 2 * C), lambda i: (i, 0)),
            pl.BlockSpec((BX, 2 * C), lambda i: (i, 0)),
            pl.BlockSpec((BX, 2 * C), lambda i: (i, 0)),
        ],
        out_shape=[
            jax.ShapeDtypeStruct((N, 2 * C), jnp.float32),
            jax.ShapeDtypeStruct((N, 2 * C), jnp.float32),
            jax.ShapeDtypeStruct((N, 2 * C), jnp.float32),
        ],
    )(x, wcat, vcat, bcat)

    p1 = pl.pallas_call(
        _stage1_kernel,
        grid=(N // BM,),
        in_specs=[
            pl.BlockSpec((BM, N), lambda i: (i, 0)),
            pl.BlockSpec((N, 2 * C), lambda i: (0, 0)),
            pl.BlockSpec((BM, 2 * C), lambda i: (i, 0)),
            pl.BlockSpec((C, C), lambda i: (0, 0)),
            pl.BlockSpec((C, C), lambda i: (0, 0)),
        ],
        out_specs=pl.BlockSpec((BM, 2 * C), lambda i: (i, 0)),
        out_shape=jax.ShapeDtypeStruct((N, 2 * C), jnp.float32),
    )(fltr, p0, s0, W01, W11)

    out = pl.pallas_call(
        _stage2_kernel,
        grid=(N // BM,),
        in_specs=[
            pl.BlockSpec((BM, N), lambda i: (i, 0)),
            pl.BlockSpec((N, 2 * C), lambda i: (0, 0)),
            pl.BlockSpec((BM, 2 * C), lambda i: (i, 0)),
        ],
        out_specs=pl.BlockSpec((BM, C), lambda i: (i, 0)),
        out_shape=jax.ShapeDtypeStruct((N, C), jnp.float32),
    )(fltr, p1, s1)

    return out
